# baseline (speedup 1.0000x reference)
import functools

import jax
import jax.numpy as jnp
from jax import lax
from jax.experimental import pallas as pl
from jax.experimental.pallas import tpu as pltpu
from jax.experimental.pallas import tpu_sc as plsc

N_NODES = 10000
K = 32
D = 128

NC = 2
NS = 16
NW = NC * NS

NPAD = 10240
PER_W = NPAD // NW
CHUNK = 4
N_CHUNKS = PER_W // CHUNK
NBUF = 4
N_GROUPS = N_CHUNKS // NBUF
IDX_ROWS = PER_W * K // D


def _mm_body(x_ref, w_ref, z_ref):
    z_ref[...] = lax.dot_general(
        x_ref[...], w_ref[...],
        dimension_numbers=(((1,), (1,)), ((), ())),
        preferred_element_type=jnp.float32,
    )


def _matmul(x_pad, w):
    blk = 512
    grid = NPAD // blk
    return pl.pallas_call(
        _mm_body,
        grid=(grid,),
        in_specs=[
            pl.BlockSpec((blk, D), lambda i: (i, 0)),
            pl.BlockSpec((D, D), lambda i: (0, 0)),
        ],
        out_specs=pl.BlockSpec((blk, D), lambda i: (i, 0)),
        out_shape=jax.ShapeDtypeStruct((NPAD, D), jnp.float32),
    )(x_pad, w)


def _reduce_chunk(gat, own_all, out_v, t, urow):
    nd = [(n, d) for n in range(CHUNK) for d in range(D // 16)]

    def add_j(j, accs):
        return tuple(
            acc + gat[n * K + j, pl.ds(d * 16, 16)]
            for acc, (n, d) in zip(accs, nd)
        )

    init = tuple(gat[n * K, pl.ds(d * 16, 16)] for (n, d) in nd)
    accs = lax.fori_loop(1, K, add_j, init)
    for acc, (n, d) in zip(accs, nd):
        sl = pl.ds(d * 16, 16)
        val = own_all[t * CHUNK + n, sl] - acc * (1.0 / K)
        out_v[urow + n, sl] = jnp.maximum(val, 0.0)


def _sc_body(z_hbm, nidx_hbm, out_hbm,
             idx_v, g0, g1, g2, g3, own_all, u01, u23,
             s0, s1, s2, s3, su01, su23, sem_own):
    wid = lax.axis_index("s") * NC + lax.axis_index("c")
    node_base = wid * PER_W
    gbufs = (g0, g1, g2, g3)
    gsems = (s0, s1, s2, s3)

    own_cp = pltpu.async_copy(z_hbm.at[pl.ds(node_base, PER_W)], own_all,
                              sem_own)
    pltpu.sync_copy(nidx_hbm.at[pl.ds(wid * IDX_ROWS, IDX_ROWS)], idx_v)

    def gather(t, g, sem):
        tc = jnp.minimum(t, N_CHUNKS - 1)
        pltpu.async_copy(z_hbm.at[idx_v.at[tc]], g, sem)

    def wait_gather(g, sem):
        pltpu.make_async_copy(z_hbm.at[pl.ds(0, D)], g, sem).wait()

    for b in range(NBUF):
        gather(b, gbufs[b], gsems[b])
    own_cp.wait()

    def loop_body(grp, _):
        t0 = NBUF * grp
        for b in range(NBUF):
            t = t0 + b
            u, su, urow = (u01, su01, b * CHUNK) if b < 2 else \
                          (u23, su23, (b - 2) * CHUNK)
            if b % 2 == 0:
                @pl.when(grp >= 1)
                def _drain():
                    pltpu.make_async_copy(
                        u, out_hbm.at[pl.ds(0, 2 * CHUNK)], su).wait()
            wait_gather(gbufs[b], gsems[b])
            _reduce_chunk(gbufs[b], own_all, u, t, urow)
            gather(t + NBUF, gbufs[b], gsems[b])
            if b % 2 == 1:
                base = node_base + (t - 1) * CHUNK
                pltpu.async_copy(u, out_hbm.at[pl.ds(base, 2 * CHUNK)], su)
        return _

    lax.fori_loop(0, N_GROUPS, loop_body, None)
    for b in range(NBUF):
        wait_gather(gbufs[b], gsems[b])
    pltpu.make_async_copy(u01, out_hbm.at[pl.ds(0, 2 * CHUNK)], su01).wait()
    pltpu.make_async_copy(u23, out_hbm.at[pl.ds(0, 2 * CHUNK)], su23).wait()


_sc_call = functools.partial(
    pl.kernel,
    out_type=jax.ShapeDtypeStruct((NPAD, D), jnp.float32),
    mesh=plsc.VectorSubcoreMesh(core_axis_name="c", subcore_axis_name="s"),
    scratch_types=[
        pltpu.VMEM((IDX_ROWS, D), jnp.int32),
        pltpu.VMEM((D, D), jnp.float32),
        pltpu.VMEM((D, D), jnp.float32),
        pltpu.VMEM((D, D), jnp.float32),
        pltpu.VMEM((D, D), jnp.float32),
        pltpu.VMEM((PER_W, D), jnp.float32),
        pltpu.VMEM((2 * CHUNK, D), jnp.float32),
        pltpu.VMEM((2 * CHUNK, D), jnp.float32),
        pltpu.SemaphoreType.DMA,
        pltpu.SemaphoreType.DMA,
        pltpu.SemaphoreType.DMA,
        pltpu.SemaphoreType.DMA,
        pltpu.SemaphoreType.DMA,
        pltpu.SemaphoreType.DMA,
        pltpu.SemaphoreType.DMA,
    ],
)(_sc_body)


def kernel(X, neigh_idx, W):
    x_pad = jnp.zeros((NPAD, D), jnp.float32).at[:N_NODES].set(X)
    nidx_pad = jnp.zeros((NPAD, K), jnp.int32).at[:N_NODES].set(neigh_idx)
    nidx2d = nidx_pad.reshape(NPAD * K // D, D)
    z = _matmul(x_pad, W)
    out = _sc_call(z, nidx2d)
    return out[:N_NODES]


# speedup vs baseline: 1.6815x; 1.6815x over previous
import functools

import numpy as np

import jax
import jax.numpy as jnp
from jax import lax
from jax.experimental import pallas as pl
from jax.experimental.pallas import tpu as pltpu
from jax.experimental.pallas import tpu_sc as plsc

N_NODES = 10000
K = 32
D = 128
DW = D // 2

NC = 2
NS = 16
NW = NC * NS

NPAD = 10240
PER_W = NPAD // NW
CHUNK = 4
N_CHUNKS = PER_W // CHUNK
NBUF = 4
N_GROUPS = N_CHUNKS // NBUF
IDX_ROWS = PER_W * K // D

_PERM = np.empty((D,), dtype=np.int32)
for _p in range(D // 32):
    for _k in range(16):
        _PERM[32 * _p + 2 * _k] = 32 * _p + _k
        _PERM[32 * _p + 2 * _k + 1] = 32 * _p + 16 + _k


def _mm_body(x_ref, w_ref, z_ref):
    z_ref[...] = lax.dot_general(
        x_ref[...], w_ref[...],
        dimension_numbers=(((1,), (1,)), ((), ())),
        preferred_element_type=jnp.float32,
    ).astype(jnp.bfloat16)


def _matmul_bf16(x_pad, wp):
    blk = 512
    grid = NPAD // blk
    return pl.pallas_call(
        _mm_body,
        grid=(grid,),
        in_specs=[
            pl.BlockSpec((blk, D), lambda i: (i, 0)),
            pl.BlockSpec((D, D), lambda i: (0, 0)),
        ],
        out_specs=pl.BlockSpec((blk, D), lambda i: (i, 0)),
        out_shape=jax.ShapeDtypeStruct((NPAD, D), jnp.bfloat16),
    )(x_pad, wp)


def _unpack2(w):
    lo = lax.bitcast_convert_type(lax.shift_left(w, 16), jnp.float32)
    hi = lax.bitcast_convert_type(lax.bitwise_and(w, jnp.int32(-65536)),
                                  jnp.float32)
    return lo, hi


def _reduce_chunk(gat, own_v, out_v, t, urow):
    inv_k = jnp.float32(1.0 / K)
    zero = jnp.float32(0.0)
    for n in range(CHUNK):
        for p in range(DW // 16):
            sl = pl.ds(p * 16, 16)
            acc_lo, acc_hi = _unpack2(gat[n * K, sl])
            for j in range(1, K):
                lo, hi = _unpack2(gat[n * K + j, sl])
                acc_lo = acc_lo + lo
                acc_hi = acc_hi + hi
            own_lo, own_hi = _unpack2(own_v[t * CHUNK + n, sl])
            out_v[urow + n, pl.ds(32 * p, 16)] = jnp.maximum(
                own_lo - acc_lo * inv_k, zero)
            out_v[urow + n, pl.ds(32 * p + 16, 16)] = jnp.maximum(
                own_hi - acc_hi * inv_k, zero)


def _sc_body(z_hbm, nidx_hbm, out_hbm,
             idx_v, g0, g1, g2, g3, own_v, u_all,
             s0, s1, s2, s3, su, sem_own):
    wid = lax.axis_index("s") * NC + lax.axis_index("c")
    node_base = wid * PER_W
    gbufs = (g0, g1, g2, g3)
    gsems = (s0, s1, s2, s3)

    own_cp = pltpu.async_copy(z_hbm.at[pl.ds(node_base, PER_W)], own_v,
                              sem_own)
    pltpu.sync_copy(nidx_hbm.at[pl.ds(wid * IDX_ROWS, IDX_ROWS)], idx_v)

    def gather(t, g, sem):
        tc = jnp.minimum(t, N_CHUNKS - 1)
        pltpu.async_copy(z_hbm.at[idx_v.at[tc]], g, sem)

    def wait_gather(g, sem):
        pltpu.make_async_copy(z_hbm.at[pl.ds(0, D)], g, sem).wait()

    for b in range(NBUF):
        gather(b, gbufs[b], gsems[b])
    own_cp.wait()

    def loop_body(grp, _):
        t0 = NBUF * grp
        @pl.when(grp >= 1)
        def _drain():
            pltpu.make_async_copy(
                u_all, out_hbm.at[pl.ds(0, NBUF * CHUNK)], su).wait()
        for b in range(NBUF):
            t = t0 + b
            wait_gather(gbufs[b], gsems[b])
            _reduce_chunk(gbufs[b], own_v, u_all, t, b * CHUNK)
            gather(t + NBUF, gbufs[b], gsems[b])
        pltpu.async_copy(
            u_all, out_hbm.at[pl.ds(node_base + t0 * CHUNK, NBUF * CHUNK)], su)
        return _

    lax.fori_loop(0, N_GROUPS, loop_body, None)
    for b in range(NBUF):
        wait_gather(gbufs[b], gsems[b])
    pltpu.make_async_copy(u_all, out_hbm.at[pl.ds(0, NBUF * CHUNK)], su).wait()


_sc_call = functools.partial(
    pl.kernel,
    out_type=jax.ShapeDtypeStruct((NPAD, D), jnp.float32),
    mesh=plsc.VectorSubcoreMesh(core_axis_name="c", subcore_axis_name="s"),
    compiler_params=pltpu.CompilerParams(use_tc_tiling_on_sc=False),
    scratch_types=[
        pltpu.VMEM((IDX_ROWS, D), jnp.int32),
        pltpu.VMEM((D, DW), jnp.int32),
        pltpu.VMEM((D, DW), jnp.int32),
        pltpu.VMEM((D, DW), jnp.int32),
        pltpu.VMEM((D, DW), jnp.int32),
        pltpu.VMEM((PER_W, DW), jnp.int32),
        pltpu.VMEM((NBUF * CHUNK, D), jnp.float32),
        pltpu.SemaphoreType.DMA,
        pltpu.SemaphoreType.DMA,
        pltpu.SemaphoreType.DMA,
        pltpu.SemaphoreType.DMA,
        pltpu.SemaphoreType.DMA,
        pltpu.SemaphoreType.DMA,
    ],
)(_sc_body)


def kernel(X, neigh_idx, W):
    x_pad = jnp.zeros((NPAD, D), jnp.float32).at[:N_NODES].set(X)
    nidx_pad = jnp.zeros((NPAD, K), jnp.int32).at[:N_NODES].set(neigh_idx)
    nidx2d = nidx_pad.reshape(NPAD * K // D, D)
    wp = W[jnp.asarray(_PERM)]
    zb = _matmul_bf16(x_pad, wp)
    zi = lax.bitcast_convert_type(zb.reshape(NPAD, DW, 2), jnp.int32)
    out = _sc_call(zi, nidx2d)
    return out[:N_NODES]
